# baseline (speedup 1.0000x reference)
import jax
import jax.numpy as jnp
from jax.experimental import pallas as pl
from jax.experimental.pallas import tpu as pltpu

_N = 10000
_F = 128
_BM = 400
_KMAIN = 9984
_KTAIL = _N - _KMAIN
_NSTEPS = _N // _BM


def _gnn_kernel(active_ref, features_ref, w_ref, adj_main_ref, adj_any_ref,
                out_ref, support_ref, tail_ref, tail_out_ref, sem_ref):
    i = pl.program_id(0)

    @pl.when(i == 0)
    def _():
        cp = pltpu.make_async_copy(
            adj_any_ref.at[:, pl.ds(_KMAIN, _KTAIL)], tail_ref, sem_ref)
        cp.start()
        s = jnp.dot(features_ref[...], w_ref[...],
                    preferred_element_type=jnp.float32)
        support_ref[...] = jnp.where(active_ref[0] != 0, jnp.tanh(s), s)
        cp.wait()
        tail_out_ref[...] = jnp.dot(
            tail_ref[...], support_ref[pl.ds(_KMAIN, _KTAIL), :],
            preferred_element_type=jnp.float32)

    out_ref[...] = (
        jnp.dot(adj_main_ref[...], support_ref[pl.ds(0, _KMAIN), :],
                preferred_element_type=jnp.float32)
        + tail_out_ref[pl.ds(i * _BM, _BM), :])


def kernel(features, adj, W, active):
    active_arr = jnp.asarray(active, jnp.int32).reshape((1,))
    return pl.pallas_call(
        _gnn_kernel,
        grid_spec=pltpu.PrefetchScalarGridSpec(
            num_scalar_prefetch=1,
            grid=(_NSTEPS,),
            in_specs=[
                pl.BlockSpec((_N, _F), lambda i, a: (0, 0)),
                pl.BlockSpec((_F, _F), lambda i, a: (0, 0)),
                pl.BlockSpec((_BM, _KMAIN), lambda i, a: (i, 0)),
                pl.BlockSpec(memory_space=pl.ANY),
            ],
            out_specs=pl.BlockSpec((_BM, _F), lambda i, a: (i, 0)),
            scratch_shapes=[
                pltpu.VMEM((_N, _F), jnp.float32),
                pltpu.VMEM((_N, _KTAIL), jnp.float32),
                pltpu.VMEM((_N, _F), jnp.float32),
                pltpu.SemaphoreType.DMA,
            ],
        ),
        out_shape=jax.ShapeDtypeStruct((_N, _F), jnp.float32),
        compiler_params=pltpu.CompilerParams(
            dimension_semantics=("arbitrary",),
        ),
    )(active_arr, features, W, adj, adj)


# speedup vs baseline: 1.0273x; 1.0273x over previous
import jax
import jax.numpy as jnp
from jax.experimental import pallas as pl
from jax.experimental.pallas import tpu as pltpu

_N = 10000
_F = 128
_BM = 400


def _gnn_kernel(active_ref, features_ref, w_ref, adj_ref, out_ref, support_ref):
    i = pl.program_id(0)

    @pl.when(i == 0)
    def _():
        s = jnp.dot(features_ref[...], w_ref[...],
                    preferred_element_type=jnp.float32)
        support_ref[...] = jnp.where(active_ref[0] != 0, jnp.tanh(s), s)

    out_ref[...] = jnp.dot(adj_ref[...], support_ref[...],
                           preferred_element_type=jnp.float32)


def kernel(features, adj, W, active):
    active_arr = jnp.asarray(active, jnp.int32).reshape((1,))
    return pl.pallas_call(
        _gnn_kernel,
        grid_spec=pltpu.PrefetchScalarGridSpec(
            num_scalar_prefetch=1,
            grid=(_N // _BM,),
            in_specs=[
                pl.BlockSpec((_N, _F), lambda i, a: (0, 0)),
                pl.BlockSpec((_F, _F), lambda i, a: (0, 0)),
                pl.BlockSpec((_BM, _N), lambda i, a: (i, 0)),
            ],
            out_specs=pl.BlockSpec((_BM, _F), lambda i, a: (i, 0)),
            scratch_shapes=[pltpu.VMEM((_N, _F), jnp.float32)],
        ),
        out_shape=jax.ShapeDtypeStruct((_N, _F), jnp.float32),
        compiler_params=pltpu.CompilerParams(
            dimension_semantics=("arbitrary",),
        ),
    )(active_arr, features, W, adj)


# speedup vs baseline: 1.0299x; 1.0025x over previous
import jax
import jax.numpy as jnp
from jax.experimental import pallas as pl
from jax.experimental.pallas import tpu as pltpu

_N = 10000
_F = 128
_CH = 400
_DEPTH = 2
_NCH = _N // _CH


def _gnn_kernel(active_ref, features_ref, w_ref, adj_ref, out_ref,
                support_ref, buf_ref, sem_ref):
    i = pl.program_id(0)

    def _start(c, slot):
        pltpu.make_async_copy(
            adj_ref.at[pl.ds(c * _CH, _CH), :],
            buf_ref.at[slot],
            sem_ref.at[slot],
        ).start()

    @pl.when(i == 0)
    def _():
        for d in range(_DEPTH):
            _start(d, d)
        s = jnp.dot(features_ref[...], w_ref[...],
                    preferred_element_type=jnp.float32)
        support_ref[...] = jnp.where(active_ref[0] != 0, jnp.tanh(s), s)

    c_next = i + _DEPTH

    @pl.when((i > 0) & (c_next - 1 < _NCH))
    def _():
        _start(c_next - 1, jax.lax.rem(c_next - 1, _DEPTH))

    slot = jax.lax.rem(i, _DEPTH)
    pltpu.make_async_copy(
        adj_ref.at[pl.ds(i * _CH, _CH), :],
        buf_ref.at[slot],
        sem_ref.at[slot],
    ).wait()
    out_ref[...] = jnp.dot(buf_ref[slot], support_ref[...],
                           preferred_element_type=jnp.float32)


def kernel(features, adj, W, active):
    active_arr = jnp.asarray(active, jnp.int32).reshape((1,))
    return pl.pallas_call(
        _gnn_kernel,
        grid_spec=pltpu.PrefetchScalarGridSpec(
            num_scalar_prefetch=1,
            grid=(_NCH,),
            in_specs=[
                pl.BlockSpec((_N, _F), lambda i, a: (0, 0)),
                pl.BlockSpec((_F, _F), lambda i, a: (0, 0)),
                pl.BlockSpec(memory_space=pl.ANY),
            ],
            out_specs=pl.BlockSpec((_CH, _F), lambda i, a: (i, 0)),
            scratch_shapes=[
                pltpu.VMEM((_N, _F), jnp.float32),
                pltpu.VMEM((_DEPTH, _CH, _N), jnp.float32),
                pltpu.SemaphoreType.DMA((_DEPTH,)),
            ],
        ),
        out_shape=jax.ShapeDtypeStruct((_N, _F), jnp.float32),
        compiler_params=pltpu.CompilerParams(
            dimension_semantics=("arbitrary",),
        ),
    )(active_arr, features, W, adj)


# speedup vs baseline: 1.0383x; 1.0082x over previous
import jax
import jax.numpy as jnp
from jax.experimental import pallas as pl
from jax.experimental.pallas import tpu as pltpu

_N = 10000
_F = 128
_BM = 400


def _gnn_kernel(active_ref, features_ref, w_ref, adj_ref, out_ref, support_ref):
    i = pl.program_id(0)

    @pl.when(i == 0)
    def _():
        s = jnp.dot(features_ref[...], w_ref[...],
                    preferred_element_type=jnp.float32)
        support_ref[...] = jnp.where(active_ref[0] != 0, jnp.tanh(s), s)

    out_ref[...] = jnp.dot(adj_ref[...], support_ref[...],
                           preferred_element_type=jnp.float32)


def kernel(features, adj, W, active):
    active_arr = jnp.asarray(active, jnp.int32).reshape((1,))
    return pl.pallas_call(
        _gnn_kernel,
        grid_spec=pltpu.PrefetchScalarGridSpec(
            num_scalar_prefetch=1,
            grid=(_N // _BM,),
            in_specs=[
                pl.BlockSpec((_N, _F), lambda i, a: (0, 0)),
                pl.BlockSpec((_F, _F), lambda i, a: (0, 0)),
                pl.BlockSpec((_BM, _N), lambda i, a: (i, 0)),
            ],
            out_specs=pl.BlockSpec((_BM, _F), lambda i, a: (i, 0)),
            scratch_shapes=[pltpu.VMEM((_N, _F), jnp.float32)],
        ),
        out_shape=jax.ShapeDtypeStruct((_N, _F), jnp.float32),
        compiler_params=pltpu.CompilerParams(
            dimension_semantics=("arbitrary",),
        ),
    )(active_arr, features, W, adj)
